# baseline (speedup 1.0000x reference)
import functools

import jax
import jax.numpy as jnp
from jax import lax
from jax.experimental import pallas as pl
from jax.experimental.pallas import tpu as pltpu
from jax.experimental.pallas import tpu_sc as plsc


def _rowsum_body(wv_ref, out_ref, *, inv_l, blk_rows):
    x = wv_ref[...]
    lane = lax.broadcasted_iota(jnp.int32, (1, 128), 1)
    a_lo = jnp.where(lane < 64, inv_l, 0.0).astype(jnp.float32)
    a_hi = jnp.where(lane >= 64, inv_l, 0.0).astype(jnp.float32)
    lo_parts, hi_parts = [], []
    for c in range(blk_rows // 128):
        xc = x[c * 128:(c + 1) * 128, :]
        dn = (((1,), (1,)), ((), ()))
        lo_parts.append(lax.dot_general(a_lo, xc, dn,
                                        preferred_element_type=jnp.float32))
        hi_parts.append(lax.dot_general(a_hi, xc, dn,
                                        preferred_element_type=jnp.float32))
    out_ref[...] = jnp.stack([jnp.concatenate(lo_parts, axis=0),
                              jnp.concatenate(hi_parts, axis=0)], axis=0)


def _rowsum(word_vectors, L):
    voc, d = word_vectors.shape
    flat = word_vectors.reshape(voc * d // 128, 128)
    blk_rows = 8192
    grid = pl.cdiv(flat.shape[0], blk_rows)
    out_rows = 4096
    assert grid * (blk_rows // 128) <= out_rows
    return pl.pallas_call(
        functools.partial(_rowsum_body, inv_l=1.0 / float(L),
                          blk_rows=blk_rows),
        grid=(grid,),
        in_specs=[pl.BlockSpec((blk_rows, 128), lambda i: (i, 0))],
        out_specs=pl.BlockSpec((2, blk_rows // 128, 128), lambda i: (0, i, 0)),
        out_shape=jax.ShapeDtypeStruct((2, out_rows, 128), jnp.float32),
    )(flat)


def _make_sc_gather(n_idx):
    info = plsc.get_sparse_core_info()
    nc, ns = info.num_cores, info.num_subcores
    nw = nc * ns
    assert n_idx % nw == 0
    per_w = n_idx // nw
    mesh = plsc.VectorSubcoreMesh(core_axis_name="c", subcore_axis_name="s")

    @functools.partial(
        pl.kernel,
        out_type=jax.ShapeDtypeStruct((n_idx,), jnp.float32),
        mesh=mesh,
        scratch_types=[
            pltpu.VMEM((per_w,), jnp.int32),
            pltpu.VMEM((per_w,), jnp.float32),
            pltpu.SemaphoreType.DMA,
        ],
    )
    def gather_k(rowsum_hbm, idx_hbm, out_hbm, idx_v, val_v, sem):
        wid = lax.axis_index("s") * nc + lax.axis_index("c")
        base = wid * per_w
        pltpu.sync_copy(idx_hbm.at[pl.ds(base, per_w)], idx_v)

        def remap(i, carry):
            v = idx_v[pl.ds(i * 16, 16)]
            idx_v[pl.ds(i * 16, 16)] = ((v & 1) << 19) | (v >> 1)
            return carry

        lax.fori_loop(0, per_w // 16, remap, 0)
        pltpu.async_copy(rowsum_hbm.at[idx_v], val_v, sem).wait()
        pltpu.sync_copy(val_v, out_hbm.at[pl.ds(base, per_w)])

    return gather_k


def _finish_body(g_ref, w_ref, b_ref, out_ref):
    s = jnp.sum(g_ref[...], axis=1)
    wsum = jnp.sum(w_ref[...], axis=1)
    out_ref[...] = s[:, None] * wsum[None, :] + b_ref[...]


def _finish(gathered, W, b):
    bsz, L = gathered.shape
    n_labels, d = W.shape
    blk_b = 1024
    return pl.pallas_call(
        _finish_body,
        grid=(bsz // blk_b,),
        in_specs=[
            pl.BlockSpec((blk_b, L), lambda i: (i, 0)),
            pl.BlockSpec((n_labels, d), lambda i: (0, 0)),
            pl.BlockSpec((1, n_labels), lambda i: (0, 0)),
        ],
        out_specs=pl.BlockSpec((blk_b, n_labels), lambda i: (i, 0)),
        out_shape=jax.ShapeDtypeStruct((bsz, n_labels), jnp.float32),
    )(gathered, W, b.reshape(1, n_labels))


def kernel(sentences, word_vectors, W, b):
    bsz, L = sentences.shape
    rowsum = _rowsum(word_vectors, L).reshape(-1)
    idx = sentences.reshape(-1).astype(jnp.int32)
    vals = _make_sc_gather(bsz * L)(rowsum, idx)
    return _finish(vals.reshape(bsz, L), W, b)


# speedup vs baseline: 4.6824x; 4.6824x over previous
import functools

import jax
import jax.numpy as jnp
from jax import lax
from jax.experimental import pallas as pl
from jax.experimental.pallas import tpu as pltpu
from jax.experimental.pallas import tpu_sc as plsc


def _rowsum_body(wvt_ref, out_ref, *, inv_l, cols):
    x = wvt_ref[...]
    s = jnp.sum(x, axis=0) * inv_l
    out_ref[...] = s.reshape(cols // 128, 128)


def _rowsum(word_vectors, L):
    voc, d = word_vectors.shape
    wvt = word_vectors.T
    cols = 32768
    grid = pl.cdiv(voc, cols)
    out_rows = 8192
    assert grid * (cols // 128) <= out_rows
    return pl.pallas_call(
        functools.partial(_rowsum_body, inv_l=1.0 / float(L), cols=cols),
        grid=(grid,),
        in_specs=[pl.BlockSpec((d, cols), lambda i: (0, i))],
        out_specs=pl.BlockSpec((cols // 128, 128), lambda i: (i, 0)),
        out_shape=jax.ShapeDtypeStruct((out_rows, 128), jnp.float32),
    )(wvt)


def _make_sc_gather(n_idx):
    info = plsc.get_sparse_core_info()
    nc, ns = info.num_cores, info.num_subcores
    nw = nc * ns
    assert n_idx % nw == 0
    per_w = n_idx // nw
    mesh = plsc.VectorSubcoreMesh(core_axis_name="c", subcore_axis_name="s")

    @functools.partial(
        pl.kernel,
        out_type=jax.ShapeDtypeStruct((n_idx,), jnp.float32),
        mesh=mesh,
        scratch_types=[
            pltpu.VMEM((per_w,), jnp.int32),
            pltpu.VMEM((per_w,), jnp.float32),
            pltpu.SemaphoreType.DMA,
        ],
    )
    def gather_k(rowsum_hbm, idx_hbm, out_hbm, idx_v, val_v, sem):
        wid = lax.axis_index("s") * nc + lax.axis_index("c")
        base = wid * per_w
        pltpu.sync_copy(idx_hbm.at[pl.ds(base, per_w)], idx_v)
        pltpu.async_copy(rowsum_hbm.at[idx_v], val_v, sem).wait()
        pltpu.sync_copy(val_v, out_hbm.at[pl.ds(base, per_w)])

    return gather_k


def _finish_body(g_ref, w_ref, b_ref, out_ref):
    s = jnp.sum(g_ref[...], axis=1)
    wsum = jnp.sum(w_ref[...], axis=1)
    out_ref[...] = s[:, None] * wsum[None, :] + b_ref[...]


def _finish(gathered, W, b):
    bsz, L = gathered.shape
    n_labels, d = W.shape
    blk_b = 1024
    return pl.pallas_call(
        _finish_body,
        grid=(bsz // blk_b,),
        in_specs=[
            pl.BlockSpec((blk_b, L), lambda i: (i, 0)),
            pl.BlockSpec((n_labels, d), lambda i: (0, 0)),
            pl.BlockSpec((1, n_labels), lambda i: (0, 0)),
        ],
        out_specs=pl.BlockSpec((blk_b, n_labels), lambda i: (i, 0)),
        out_shape=jax.ShapeDtypeStruct((bsz, n_labels), jnp.float32),
    )(gathered, W, b.reshape(1, n_labels))


def kernel(sentences, word_vectors, W, b):
    bsz, L = sentences.shape
    rowsum = _rowsum(word_vectors, L).reshape(-1)
    idx = sentences.reshape(-1).astype(jnp.int32)
    vals = _make_sc_gather(bsz * L)(rowsum, idx)
    return _finish(vals.reshape(bsz, L), W, b)


# speedup vs baseline: 5.4912x; 1.1727x over previous
import functools

import jax
import jax.numpy as jnp
from jax import lax
from jax.experimental import pallas as pl
from jax.experimental.pallas import tpu as pltpu
from jax.experimental.pallas import tpu_sc as plsc


def _rowsum_body(wvt_ref, out_ref, *, inv_l, cols):
    x = wvt_ref[...]
    s = jnp.sum(x, axis=0) * inv_l
    out_ref[...] = s.reshape(cols // 128, 128)


def _rowsum(word_vectors, L):
    voc, d = word_vectors.shape
    wvt = word_vectors.T
    cols = 32768
    grid = pl.cdiv(voc, cols)
    out_rows = 8192
    assert grid * (cols // 128) <= out_rows
    return pl.pallas_call(
        functools.partial(_rowsum_body, inv_l=1.0 / float(L), cols=cols),
        grid=(grid,),
        in_specs=[pl.BlockSpec((d, cols), lambda i: (0, i))],
        out_specs=pl.BlockSpec((cols // 128, 128), lambda i: (i, 0)),
        out_shape=jax.ShapeDtypeStruct((out_rows, 128), jnp.float32),
    )(wvt)


def _make_sc_gather(bsz, L):
    info = plsc.get_sparse_core_info()
    nc, ns = info.num_cores, info.num_subcores
    nw = nc * ns
    n_idx = bsz * L
    assert n_idx % nw == 0 and bsz % (nw * 8) == 0
    per_s = bsz // nw
    mesh = plsc.VectorSubcoreMesh(core_axis_name="c", subcore_axis_name="s")

    @functools.partial(
        pl.kernel,
        out_type=jax.ShapeDtypeStruct((n_idx,), jnp.float32),
        mesh=mesh,
        scratch_types=[
            pltpu.VMEM((L * per_s,), jnp.int32),
            pltpu.VMEM((L * per_s,), jnp.float32),
            pltpu.SemaphoreType.DMA,
            pltpu.SemaphoreType.DMA,
            pltpu.SemaphoreType.DMA,
        ],
    )
    def gather_k(rowsum_hbm, idx_hbm, out_hbm, idx_v, val_v, sem_i, sem_g,
                 sem_o):
        wid = lax.axis_index("s") * nc + lax.axis_index("c")
        base = wid * per_s
        ld = [pltpu.async_copy(idx_hbm.at[pl.ds(l * bsz + base, per_s)],
                               idx_v.at[pl.ds(l * per_s, per_s)], sem_i)
              for l in range(L)]
        for h in ld:
            h.wait()
        pltpu.async_copy(rowsum_hbm.at[idx_v], val_v, sem_g).wait()
        st = [pltpu.async_copy(val_v.at[pl.ds(l * per_s, per_s)],
                               out_hbm.at[pl.ds(l * bsz + base, per_s)],
                               sem_o) for l in range(L)]
        for h in st:
            h.wait()

    return gather_k


def _finish_body(v_ref, w_ref, b_ref, out_ref, *, blk_b, L, qrows):
    p = pl.program_id(0)
    rows_per_l = v_ref.shape[0] // L
    acc = jnp.zeros((qrows, 128), jnp.float32)
    for l in range(L):
        acc = acc + v_ref[pl.ds(l * rows_per_l + qrows * p, qrows), :]
    ones = jnp.ones((1, w_ref.shape[1]), jnp.float32)
    wsum_row = lax.dot_general(ones, w_ref[...], (((1,), (1,)), ((), ())),
                               preferred_element_type=jnp.float32)
    parts = []
    for q in range(qrows):
        s_row = acc[q:q + 1, :]
        parts.append(lax.dot_general(
            s_row, wsum_row, (((0,), (0,)), ((), ())),
            preferred_element_type=jnp.float32) + b_ref[...])
    out_ref[...] = jnp.concatenate(parts, axis=0)


def _finish(vals, bsz, L, W, b):
    n_labels, d = W.shape
    blk_b = 1024
    qrows = blk_b // 128
    v2d = vals.reshape(bsz * L // 128, 128)
    return pl.pallas_call(
        functools.partial(_finish_body, blk_b=blk_b, L=L, qrows=qrows),
        grid=(bsz // blk_b,),
        in_specs=[
            pl.BlockSpec(v2d.shape, lambda i: (0, 0)),
            pl.BlockSpec((n_labels, d), lambda i: (0, 0)),
            pl.BlockSpec((1, n_labels), lambda i: (0, 0)),
        ],
        out_specs=pl.BlockSpec((blk_b, n_labels), lambda i: (i, 0)),
        out_shape=jax.ShapeDtypeStruct((bsz, n_labels), jnp.float32),
    )(v2d, W, b.reshape(1, n_labels))


def kernel(sentences, word_vectors, W, b):
    bsz, L = sentences.shape
    rowsum = _rowsum(word_vectors, L).reshape(-1)
    idx = sentences.T.reshape(-1).astype(jnp.int32)
    vals = _make_sc_gather(bsz, L)(rowsum, idx)
    return _finish(vals, bsz, L, W, b)
